# baseline (speedup 1.0000x reference)
import functools

import jax
import jax.numpy as jnp
from jax import lax
from jax.experimental import pallas as pl
from jax.experimental.pallas import tpu as pltpu
from jax.experimental.pallas import tpu_sc as plsc

N = 50000
E = 300000
D_RAW = 256
D = 128

NP = 50176
CH = NP // 8
ACC_ROWS = 6400
EP = 311296
NB_D = 152
NB_S = 152
GB = 19
NG = NB_S // GB
GBLK = 40
PADV = 1 << 30
BN = 3584
GRID_N = NP // BN
TS = NP // 16

_f32 = jnp.float32
_i32 = jnp.int32


def _mm_reduce_body(x_ref, w_ref, b_ref, o_ref):
    o_ref[...] = (
        jnp.dot(x_ref[...], w_ref[...], preferred_element_type=_f32)
        + b_ref[...]
    )


def _mm_reduce(x, w, b):
    return pl.pallas_call(
        _mm_reduce_body,
        grid=(GRID_N,),
        in_specs=[
            pl.BlockSpec((BN, D_RAW), lambda i: (i, 0)),
            pl.BlockSpec((D_RAW, D), lambda i: (0, 0)),
            pl.BlockSpec((1, D), lambda i: (0, 0)),
        ],
        out_specs=pl.BlockSpec((BN, D), lambda i: (i, 0)),
        out_shape=jax.ShapeDtypeStruct((NP, D), _f32),
    )(x, w, b)


def _norm_from_deg(deg):
    return jnp.where(deg > 0, lax.rsqrt(jnp.maximum(deg, 1e-12)), 0.0)


def _mm_rel_body(x_ref, w_ref, dp_ref, o_ref):
    r = pl.program_id(0)
    deg = jnp.where(
        r == 0,
        dp_ref[0, 0, :] + dp_ref[1, 0, :],
        dp_ref[0, 2, :] + dp_ref[1, 2, :],
    )
    nrm = _norm_from_deg(deg)
    o_ref[0] = jnp.dot(
        x_ref[...] * nrm[:, None], w_ref[0], preferred_element_type=_f32
    )


def _mm_rel(x, ws, degp):
    return pl.pallas_call(
        _mm_rel_body,
        grid=(2, GRID_N),
        in_specs=[
            pl.BlockSpec((BN, D), lambda r, i: (i, 0)),
            pl.BlockSpec((1, D, D), lambda r, i: (r, 0, 0)),
            pl.BlockSpec((2, 4, BN), lambda r, i: (0, 0, i)),
        ],
        out_specs=pl.BlockSpec((1, BN, D), lambda r, i: (r, i, 0)),
        out_shape=jax.ShapeDtypeStruct((2, NP, D), _f32),
    )(x, ws, degp)


def _fused_h_body(a_ref, dp_ref, b_ref, x0_ref, o_ref):
    nd0 = _norm_from_deg(dp_ref[0, 1, :] + dp_ref[1, 1, :])
    nd1 = _norm_from_deg(dp_ref[0, 3, :] + dp_ref[1, 3, :])
    h = (
        a_ref[0] * nd0[:, None]
        + a_ref[1] * nd1[:, None]
        + b_ref[0]
        + b_ref[1]
        + x0_ref[...]
    )
    o_ref[...] = jnp.maximum(h, 0.0)


def _fused_h(aggs, degp, bs, x0):
    return pl.pallas_call(
        _fused_h_body,
        grid=(GRID_N,),
        in_specs=[
            pl.BlockSpec((2, BN, D), lambda i: (0, i, 0)),
            pl.BlockSpec((2, 4, BN), lambda i: (0, 0, i)),
            pl.BlockSpec((2, 1, D), lambda i: (0, 0, 0)),
            pl.BlockSpec((BN, D), lambda i: (i, 0)),
        ],
        out_specs=pl.BlockSpec((BN, D), lambda i: (i, 0)),
        out_shape=jax.ShapeDtypeStruct((NP, D), _f32),
    )(aggs, degp, bs, x0)


def _fused_out_body(a_ref, dp_ref, b_ref, h_ref, w_ref, bi_ref, o_ref):
    nd0 = _norm_from_deg(dp_ref[0, 1, :] + dp_ref[1, 1, :])
    nd1 = _norm_from_deg(dp_ref[0, 3, :] + dp_ref[1, 3, :])
    h2 = (
        a_ref[0] * nd0[:, None]
        + a_ref[1] * nd1[:, None]
        + b_ref[0]
        + b_ref[1]
        + h_ref[...]
    )
    o_ref[...] = (
        jnp.dot(h2, w_ref[...], preferred_element_type=_f32) + bi_ref[...]
    )


def _fused_out(aggs, degp, bs, h, w_inc, b_inc):
    return pl.pallas_call(
        _fused_out_body,
        grid=(GRID_N,),
        in_specs=[
            pl.BlockSpec((2, BN, D), lambda i: (0, i, 0)),
            pl.BlockSpec((2, 4, BN), lambda i: (0, 0, i)),
            pl.BlockSpec((2, 1, D), lambda i: (0, 0, 0)),
            pl.BlockSpec((BN, D), lambda i: (i, 0)),
            pl.BlockSpec((D, D_RAW), lambda i: (0, 0)),
            pl.BlockSpec((1, D_RAW), lambda i: (0, 0)),
        ],
        out_specs=pl.BlockSpec((BN, D_RAW), lambda i: (i, 0)),
        out_shape=jax.ShapeDtypeStruct((N, D_RAW), _f32),
    )(aggs, degp, bs, h, w_inc, b_inc)


def _zero_rows(ref, nrows):
    z = jnp.zeros((16,), _f32)

    def body(i, _):
        for k in range(8):
            ref[i, pl.ds(k * 16, 16)] = z
        return 0

    lax.fori_loop(0, nrows, body, 0)


def _remap_inplace(buf, nb, fn):
    def body(j, _):
        for k in range(4):
            v = buf[j, pl.ds(k * 16, 16)]
            buf[j, pl.ds(k * 16, 16)] = fn(v)
        return 0

    lax.fori_loop(0, nb, body, 0)


def _deg_body(s0, d0, s1, d1, degp, locbuf, ones, zrow, daccs):
    c = lax.axis_index("c")
    s = lax.axis_index("s")
    wid = s * 2 + c

    def body_ones(i, _):
        ones[pl.ds(i * 16, 16)] = jnp.ones((16,), _f32)
        return 0

    lax.fori_loop(0, 4, body_ones, 0)

    def body_zrow(i, _):
        zrow[pl.ds(i * 16, 16)] = jnp.zeros((16,), _f32)
        return 0

    lax.fori_loop(0, TS // 16, body_zrow, 0)

    tbase = pl.multiple_of(s * TS, 8)
    for a in range(4):
        pltpu.sync_copy(zrow, daccs[a].at[pl.ds(tbase, TS)])
    plsc.subcore_barrier()

    nval = jnp.int32(N)
    for a, arr in enumerate((s0, d0, s1, d1)):
        pltpu.sync_copy(arr.at[wid], locbuf)
        _remap_inplace(locbuf, NB_D, lambda v: jnp.where(v < nval, v, nval))
        acc = daccs[a]

        def body(j, _):
            pltpu.sync_copy(ones, acc.at[locbuf.at[j]], add=True)
            return 0

        lax.fori_loop(0, NB_D, body, 0)

    plsc.subcore_barrier()
    for a in range(4):
        obase = pl.multiple_of(c * (4 * NP) + a * NP + s * TS, 8)
        pltpu.sync_copy(daccs[a].at[pl.ds(tbase, TS)], zrow)
        pltpu.sync_copy(zrow, degp.at[pl.ds(obase, TS)])


@functools.partial(
    pl.kernel,
    out_type=jax.ShapeDtypeStruct((2 * 4 * NP,), _f32),
    mesh=plsc.VectorSubcoreMesh(core_axis_name="c", subcore_axis_name="s",
                                num_cores=2, num_subcores=16),
    scratch_types=[
        pltpu.VMEM((NB_D, 64), _i32),
        pltpu.VMEM((64,), _f32),
        pltpu.VMEM((TS,), _f32),
        [pltpu.VMEM_SHARED((NP,), _f32) for _ in range(4)],
    ],
)
def _deg_kernel(s0, d0, s1, d1, degp, locbuf, ones, zrow, daccs):
    _deg_body(s0, d0, s1, d1, degp, locbuf, ones, zrow, daccs)


def _scatter_body(e0, e1, hwflat, aggs,
                  sdg, sidx, didx, bufa, bufb,
                  gsa, gsb, ssa, ssb, acc):
    c = lax.axis_index("c")
    s = lax.axis_index("s")

    chval = jnp.int32(CH)
    for r in range(2):
        ed = e0 if r == 0 else e1
        roff = jnp.int32(r * NP)

        for chunk in range(4):
            base = (c * 4 + chunk) * CH
            _zero_rows(bufa, 128)
            for t in range(3):
                zoff = pl.multiple_of(s * 400 + t * 128, 8)
                pltpu.sync_copy(bufa, acc.at[pl.ds(zoff, 128)])
            zoff = pl.multiple_of(s * 400 + 384, 8)
            pltpu.sync_copy(bufa.at[pl.ds(0, 16)], acc.at[pl.ds(zoff, 16)])
            plsc.subcore_barrier()

            def gbody(g, _):
                goff = pl.multiple_of(g * GBLK, 8)
                pltpu.sync_copy(ed.at[s, pl.ds(goff, GBLK)], sdg)
                for j in range(GB):
                    for k in range(8):
                        v = sdg[j, pl.ds(k * 16, 16)]
                        w = sdg[GB + 1 + j, pl.ds(k * 16, 16)]
                        m = (w >= base) & (w < base + chval)
                        sidx[j, pl.ds(k * 16, 16)] = jnp.where(
                            m, v + roff, -1
                        )
                        didx[j, pl.ds(k * 16, 16)] = jnp.where(
                            m, w - base, chval
                        )
                bufs = (bufa, bufb)
                gsems = (gsa, gsb)
                ssems = (ssa, ssb)
                gd = [None] * GB
                sd = [None] * GB
                for b in range(GB + 1):
                    if b < GB:
                        if b >= 2:
                            sd[b - 2].wait()
                        gd[b] = pltpu.async_copy(
                            hwflat.at[plsc.Indices(sidx.at[b],
                                                   ignored_value=-1)],
                            bufs[b % 2], gsems[b % 2]
                        )
                    if b >= 1:
                        gd[b - 1].wait()
                        sd[b - 1] = pltpu.async_copy(
                            bufs[(b - 1) % 2],
                            acc.at[plsc.Indices(didx.at[b - 1],
                                                ignored_value=CH)],
                            ssems[(b - 1) % 2],
                            add=True,
                        )
                sd[GB - 2].wait()
                sd[GB - 1].wait()
                return 0

            lax.fori_loop(0, NG, gbody, 0)
            plsc.subcore_barrier()

            for t in range(3):
                soff = pl.multiple_of(s * 392 + t * 128, 8)
                doff = pl.multiple_of(base + s * 392 + t * 128, 8)
                pltpu.sync_copy(acc.at[pl.ds(soff, 128)],
                                aggs.at[r, pl.ds(doff, 128)])
            soff = pl.multiple_of(s * 392 + 384, 8)
            doff = pl.multiple_of(base + s * 392 + 384, 8)
            pltpu.sync_copy(acc.at[pl.ds(soff, 8)],
                            aggs.at[r, pl.ds(doff, 8)])
            plsc.subcore_barrier()


@functools.partial(
    pl.kernel,
    out_type=jax.ShapeDtypeStruct((2, NP, D), _f32),
    mesh=plsc.VectorSubcoreMesh(core_axis_name="c", subcore_axis_name="s",
                                num_cores=2, num_subcores=16),
    scratch_types=[
        pltpu.VMEM((GBLK, 128), _i32),
        pltpu.VMEM((GB, 128), _i32),
        pltpu.VMEM((GB, 128), _i32),
        pltpu.VMEM((128, D), _f32),
        pltpu.VMEM((128, D), _f32),
        pltpu.SemaphoreType.DMA,
        pltpu.SemaphoreType.DMA,
        pltpu.SemaphoreType.DMA,
        pltpu.SemaphoreType.DMA,
        pltpu.VMEM_SHARED((ACC_ROWS, D), _f32),
    ],
)
def _scatter_kernel(e0, e1, hwflat, aggs,
                    sdg, sidx, didx, bufa, bufb,
                    gsa, gsb, ssa, ssb, acc):
    _scatter_body(e0, e1, hwflat, aggs,
                  sdg, sidx, didx, bufa, bufb,
                  gsa, gsb, ssa, ssb, acc)


def kernel(x, edge_index0, edge_index1, W_reduce, b_reduce,
           W1_0, b1_0, W1_1, b1_1, W2_0, b2_0, W2_1, b2_1, W_inc, b_inc):
    pad = jnp.full((EP - E,), PADV, _i32)
    es0 = jnp.concatenate([edge_index0[0], pad])
    ed0 = jnp.concatenate([edge_index0[1], pad])
    es1 = jnp.concatenate([edge_index1[0], pad])
    ed1 = jnp.concatenate([edge_index1[1], pad])
    s0d, d0d, s1d, d1d = (a.reshape(32, NB_D, 64) for a in (es0, ed0, es1, ed1))

    def _pack(sf, df):
        s3 = sf.reshape(16, NG, GB, 128)
        d3 = df.reshape(16, NG, GB, 128)
        z = jnp.zeros((16, NG, 1, 128), _i32)
        return jnp.concatenate([s3, z, d3, z], axis=2).reshape(16, NG * GBLK, 128)

    e0pk = _pack(es0, ed0)
    e1pk = _pack(es1, ed1)

    w1s = jnp.stack([W1_0, W1_1])
    b1s = jnp.stack([b1_0, b1_1])[:, None, :]
    w2s = jnp.stack([W2_0, W2_1])
    b2s = jnp.stack([b2_0, b2_1])[:, None, :]

    degp = _deg_kernel(s0d, d0d, s1d, d1d).reshape(2, 4, NP)

    x0 = _mm_reduce(x, W_reduce, b_reduce[None, :])
    hw1 = _mm_rel(x0, w1s, degp)
    agg1 = _scatter_kernel(e0pk, e1pk, hw1.reshape(2 * NP, D))
    h = _fused_h(agg1, degp, b1s, x0)
    hw2 = _mm_rel(h, w2s, degp)
    agg2 = _scatter_kernel(e0pk, e1pk, hw2.reshape(2 * NP, D))
    out = _fused_out(agg2, degp, b2s, h, W_inc, b_inc[None, :])
    return out


# speedup vs baseline: 1.3089x; 1.3089x over previous
import functools

import jax
import jax.numpy as jnp
from jax import lax
from jax.experimental import pallas as pl
from jax.experimental.pallas import tpu as pltpu
from jax.experimental.pallas import tpu_sc as plsc

N = 50000
E = 300000
D_RAW = 256
D = 128

NP = 50176
CH = NP // 8
ACC_ROWS = 6400
EP = 311296
NB_D = 152
NB_S = 152
GB = 19
NG = NB_S // GB
GBLK = 40
PADV = 1 << 30
BN = 3584
GRID_N = NP // BN
TS = NP // 16

_f32 = jnp.float32
_i32 = jnp.int32


def _mm_reduce_body(x_ref, w_ref, b_ref, o_ref):
    o_ref[...] = (
        jnp.dot(x_ref[...], w_ref[...], preferred_element_type=_f32)
        + b_ref[...]
    )


def _mm_reduce(x, w, b):
    return pl.pallas_call(
        _mm_reduce_body,
        grid=(GRID_N,),
        in_specs=[
            pl.BlockSpec((BN, D_RAW), lambda i: (i, 0)),
            pl.BlockSpec((D_RAW, D), lambda i: (0, 0)),
            pl.BlockSpec((1, D), lambda i: (0, 0)),
        ],
        out_specs=pl.BlockSpec((BN, D), lambda i: (i, 0)),
        out_shape=jax.ShapeDtypeStruct((NP, D), _f32),
    )(x, w, b)


def _norm_from_deg(deg):
    return jnp.where(deg > 0, lax.rsqrt(jnp.maximum(deg, 1e-12)), 0.0)


def _mm_rel_body(x_ref, w_ref, dp_ref, o_ref):
    r = pl.program_id(0)
    deg = jnp.where(
        r == 0,
        dp_ref[0, 0, :] + dp_ref[1, 0, :],
        dp_ref[0, 2, :] + dp_ref[1, 2, :],
    )
    nrm = _norm_from_deg(deg)
    o_ref[0] = jnp.dot(
        x_ref[...] * nrm[:, None], w_ref[0], preferred_element_type=_f32
    )


def _mm_rel(x, ws, degp):
    return pl.pallas_call(
        _mm_rel_body,
        grid=(2, GRID_N),
        in_specs=[
            pl.BlockSpec((BN, D), lambda r, i: (i, 0)),
            pl.BlockSpec((1, D, D), lambda r, i: (r, 0, 0)),
            pl.BlockSpec((2, 4, BN), lambda r, i: (0, 0, i)),
        ],
        out_specs=pl.BlockSpec((1, BN, D), lambda r, i: (r, i, 0)),
        out_shape=jax.ShapeDtypeStruct((2, NP, D), _f32),
    )(x, ws, degp)


def _fused_h_body(a_ref, dp_ref, b_ref, x0_ref, o_ref):
    nd0 = _norm_from_deg(dp_ref[0, 1, :] + dp_ref[1, 1, :])
    nd1 = _norm_from_deg(dp_ref[0, 3, :] + dp_ref[1, 3, :])
    h = (
        a_ref[0] * nd0[:, None]
        + a_ref[1] * nd1[:, None]
        + b_ref[0]
        + b_ref[1]
        + x0_ref[...]
    )
    o_ref[...] = jnp.maximum(h, 0.0)


def _fused_h(aggs, degp, bs, x0):
    return pl.pallas_call(
        _fused_h_body,
        grid=(GRID_N,),
        in_specs=[
            pl.BlockSpec((2, BN, D), lambda i: (0, i, 0)),
            pl.BlockSpec((2, 4, BN), lambda i: (0, 0, i)),
            pl.BlockSpec((2, 1, D), lambda i: (0, 0, 0)),
            pl.BlockSpec((BN, D), lambda i: (i, 0)),
        ],
        out_specs=pl.BlockSpec((BN, D), lambda i: (i, 0)),
        out_shape=jax.ShapeDtypeStruct((NP, D), _f32),
    )(aggs, degp, bs, x0)


def _fused_out_body(a_ref, dp_ref, b_ref, h_ref, w_ref, bi_ref, o_ref):
    nd0 = _norm_from_deg(dp_ref[0, 1, :] + dp_ref[1, 1, :])
    nd1 = _norm_from_deg(dp_ref[0, 3, :] + dp_ref[1, 3, :])
    h2 = (
        a_ref[0] * nd0[:, None]
        + a_ref[1] * nd1[:, None]
        + b_ref[0]
        + b_ref[1]
        + h_ref[...]
    )
    o_ref[...] = (
        jnp.dot(h2, w_ref[...], preferred_element_type=_f32) + bi_ref[...]
    )


def _fused_out(aggs, degp, bs, h, w_inc, b_inc):
    return pl.pallas_call(
        _fused_out_body,
        grid=(GRID_N,),
        in_specs=[
            pl.BlockSpec((2, BN, D), lambda i: (0, i, 0)),
            pl.BlockSpec((2, 4, BN), lambda i: (0, 0, i)),
            pl.BlockSpec((2, 1, D), lambda i: (0, 0, 0)),
            pl.BlockSpec((BN, D), lambda i: (i, 0)),
            pl.BlockSpec((D, D_RAW), lambda i: (0, 0)),
            pl.BlockSpec((1, D_RAW), lambda i: (0, 0)),
        ],
        out_specs=pl.BlockSpec((BN, D_RAW), lambda i: (i, 0)),
        out_shape=jax.ShapeDtypeStruct((N, D_RAW), _f32),
    )(aggs, degp, bs, h, w_inc, b_inc)


def _zero_rows(ref, nrows):
    z = jnp.zeros((16,), _f32)

    def body(i, _):
        for k in range(8):
            ref[i, pl.ds(k * 16, 16)] = z
        return 0

    lax.fori_loop(0, nrows, body, 0)


def _remap_inplace(buf, nb, fn):
    def body(j, _):
        for k in range(4):
            v = buf[j, pl.ds(k * 16, 16)]
            buf[j, pl.ds(k * 16, 16)] = fn(v)
        return 0

    lax.fori_loop(0, nb, body, 0)


def _deg_body(s0, d0, s1, d1, degp, locbuf, ones, zrow, daccs):
    c = lax.axis_index("c")
    s = lax.axis_index("s")
    wid = s * 2 + c

    def body_ones(i, _):
        ones[pl.ds(i * 16, 16)] = jnp.ones((16,), _f32)
        return 0

    lax.fori_loop(0, 4, body_ones, 0)

    def body_zrow(i, _):
        zrow[pl.ds(i * 16, 16)] = jnp.zeros((16,), _f32)
        return 0

    lax.fori_loop(0, TS // 16, body_zrow, 0)

    tbase = pl.multiple_of(s * TS, 8)
    for a in range(4):
        pltpu.sync_copy(zrow, daccs[a].at[pl.ds(tbase, TS)])
    plsc.subcore_barrier()

    nval = jnp.int32(N)
    for a, arr in enumerate((s0, d0, s1, d1)):
        pltpu.sync_copy(arr.at[wid], locbuf)
        _remap_inplace(locbuf, NB_D, lambda v: jnp.where(v < nval, v, nval))
        acc = daccs[a]

        def body(j, _):
            pltpu.sync_copy(ones, acc.at[locbuf.at[j]], add=True)
            return 0

        lax.fori_loop(0, NB_D, body, 0)

    plsc.subcore_barrier()
    for a in range(4):
        obase = pl.multiple_of(c * (4 * NP) + a * NP + s * TS, 8)
        pltpu.sync_copy(daccs[a].at[pl.ds(tbase, TS)], zrow)
        pltpu.sync_copy(zrow, degp.at[pl.ds(obase, TS)])


@functools.partial(
    pl.kernel,
    out_type=jax.ShapeDtypeStruct((2 * 4 * NP,), _f32),
    mesh=plsc.VectorSubcoreMesh(core_axis_name="c", subcore_axis_name="s",
                                num_cores=2, num_subcores=16),
    scratch_types=[
        pltpu.VMEM((NB_D, 64), _i32),
        pltpu.VMEM((64,), _f32),
        pltpu.VMEM((TS,), _f32),
        [pltpu.VMEM_SHARED((NP,), _f32) for _ in range(4)],
    ],
)
def _deg_kernel(s0, d0, s1, d1, degp, locbuf, ones, zrow, daccs):
    _deg_body(s0, d0, s1, d1, degp, locbuf, ones, zrow, daccs)


def _scatter_body(e0, e1, hwflat, aggs,
                  sdg, sidx, didx, bufa, bufb, bufc, bufd,
                  gsa, gsb, gsc, gsd, ssa, ssb, ssc, ssd, acc):
    c = lax.axis_index("c")
    s = lax.axis_index("s")

    chval = jnp.int32(CH)
    for r in range(2):
        ed = e0 if r == 0 else e1
        roff = jnp.int32(r * NP)

        for chunk in range(4):
            base = (c * 4 + chunk) * CH
            _zero_rows(bufa, 128)
            for t in range(3):
                zoff = pl.multiple_of(s * 400 + t * 128, 8)
                pltpu.sync_copy(bufa, acc.at[pl.ds(zoff, 128)])
            zoff = pl.multiple_of(s * 400 + 384, 8)
            pltpu.sync_copy(bufa.at[pl.ds(0, 16)], acc.at[pl.ds(zoff, 16)])
            plsc.subcore_barrier()

            def gbody(g, _):
                goff = pl.multiple_of(g * GBLK, 8)
                pltpu.sync_copy(ed.at[s, pl.ds(goff, GBLK)], sdg)
                for j in range(GB):
                    for k in range(8):
                        v = sdg[j, pl.ds(k * 16, 16)]
                        w = sdg[GB + 1 + j, pl.ds(k * 16, 16)]
                        m = (w >= base) & (w < base + chval)
                        sidx[j, pl.ds(k * 16, 16)] = jnp.where(
                            m, v + roff, -1
                        )
                        didx[j, pl.ds(k * 16, 16)] = jnp.where(
                            m, w - base, chval
                        )
                bufs = (bufa, bufb, bufc, bufd)
                gsems = (gsa, gsb, gsc, gsd)
                ssems = (ssa, ssb, ssc, ssd)
                nd = 4
                gd = [None] * GB
                sd = [None] * GB
                for b in range(GB + 1):
                    if b < GB:
                        if b >= nd:
                            sd[b - nd].wait()
                        gd[b] = pltpu.async_copy(
                            hwflat.at[plsc.Indices(sidx.at[b],
                                                   ignored_value=-1)],
                            bufs[b % nd], gsems[b % nd]
                        )
                    if b >= 1:
                        gd[b - 1].wait()
                        sd[b - 1] = pltpu.async_copy(
                            bufs[(b - 1) % nd],
                            acc.at[plsc.Indices(didx.at[b - 1],
                                                ignored_value=CH)],
                            ssems[(b - 1) % nd],
                            add=True,
                        )
                for b in range(max(0, GB - nd + 1), GB):
                    sd[b].wait()
                return 0

            lax.fori_loop(0, NG, gbody, 0)
            plsc.subcore_barrier()

            for t in range(3):
                soff = pl.multiple_of(s * 392 + t * 128, 8)
                doff = pl.multiple_of(base + s * 392 + t * 128, 8)
                pltpu.sync_copy(acc.at[pl.ds(soff, 128)],
                                aggs.at[r, pl.ds(doff, 128)])
            soff = pl.multiple_of(s * 392 + 384, 8)
            doff = pl.multiple_of(base + s * 392 + 384, 8)
            pltpu.sync_copy(acc.at[pl.ds(soff, 8)],
                            aggs.at[r, pl.ds(doff, 8)])
            plsc.subcore_barrier()


@functools.partial(
    pl.kernel,
    out_type=jax.ShapeDtypeStruct((2, NP, D), _f32),
    mesh=plsc.VectorSubcoreMesh(core_axis_name="c", subcore_axis_name="s",
                                num_cores=2, num_subcores=16),
    scratch_types=[
        pltpu.VMEM((GBLK, 128), _i32),
        pltpu.VMEM((GB, 128), _i32),
        pltpu.VMEM((GB, 128), _i32),
        pltpu.VMEM((128, D), _f32),
        pltpu.VMEM((128, D), _f32),
        pltpu.VMEM((128, D), _f32),
        pltpu.VMEM((128, D), _f32),
        pltpu.SemaphoreType.DMA,
        pltpu.SemaphoreType.DMA,
        pltpu.SemaphoreType.DMA,
        pltpu.SemaphoreType.DMA,
        pltpu.SemaphoreType.DMA,
        pltpu.SemaphoreType.DMA,
        pltpu.SemaphoreType.DMA,
        pltpu.SemaphoreType.DMA,
        pltpu.VMEM_SHARED((ACC_ROWS, D), _f32),
    ],
)
def _scatter_kernel(e0, e1, hwflat, aggs,
                    sdg, sidx, didx, bufa, bufb, bufc, bufd,
                    gsa, gsb, gsc, gsd, ssa, ssb, ssc, ssd, acc):
    _scatter_body(e0, e1, hwflat, aggs,
                  sdg, sidx, didx, bufa, bufb, bufc, bufd,
                  gsa, gsb, gsc, gsd, ssa, ssb, ssc, ssd, acc)


def kernel(x, edge_index0, edge_index1, W_reduce, b_reduce,
           W1_0, b1_0, W1_1, b1_1, W2_0, b2_0, W2_1, b2_1, W_inc, b_inc):
    pad = jnp.full((EP - E,), PADV, _i32)
    es0 = jnp.concatenate([edge_index0[0], pad])
    ed0 = jnp.concatenate([edge_index0[1], pad])
    es1 = jnp.concatenate([edge_index1[0], pad])
    ed1 = jnp.concatenate([edge_index1[1], pad])
    s0d, d0d, s1d, d1d = (a.reshape(32, NB_D, 64) for a in (es0, ed0, es1, ed1))

    def _pack(sf, df):
        s3 = sf.reshape(16, NG, GB, 128)
        d3 = df.reshape(16, NG, GB, 128)
        z = jnp.zeros((16, NG, 1, 128), _i32)
        return jnp.concatenate([s3, z, d3, z], axis=2).reshape(16, NG * GBLK, 128)

    e0pk = _pack(es0, ed0)
    e1pk = _pack(es1, ed1)

    w1s = jnp.stack([W1_0, W1_1])
    b1s = jnp.stack([b1_0, b1_1])[:, None, :]
    w2s = jnp.stack([W2_0, W2_1])
    b2s = jnp.stack([b2_0, b2_1])[:, None, :]

    degp = _deg_kernel(s0d, d0d, s1d, d1d).reshape(2, 4, NP)

    x0 = _mm_reduce(x, W_reduce, b_reduce[None, :])
    hw1 = _mm_rel(x0, w1s, degp)
    agg1 = _scatter_kernel(e0pk, e1pk, hw1.reshape(2 * NP, D))
    h = _fused_h(agg1, degp, b1s, x0)
    hw2 = _mm_rel(h, w2s, degp)
    agg2 = _scatter_kernel(e0pk, e1pk, hw2.reshape(2 * NP, D))
    out = _fused_out(agg2, degp, b2s, h, W_inc, b_inc[None, :])
    return out
